# Initial kernel scaffold; baseline (speedup 1.0000x reference)
import jax
import jax.numpy as jnp
from jax.experimental import pallas as pl


def kernel(wids, lens, edge_index, word_emb, W_self1, W_neigh1, b1, W_self2, W_neigh2, b2):
    raise NotImplementedError("write your pallas kernel here")


# speedup vs baseline: 3.4279x; 3.4279x over previous
import jax
import jax.numpy as jnp
from jax import lax
from jax.experimental import pallas as pl
from jax.experimental.pallas import tpu as pltpu
from jax.experimental.pallas import tpu_sc as plsc

N = 50000
E = 800000
D = 64
L = 20
H = 32
NW = 32
PT = 1568
NP = NW * PT
CA = 112
NCA = PT // CA
CB = 128
NCB = E // CB
TPS = 16
PS = NP // TPS
DGR = 98
CHN = 64
DPR = 392


def _pool_body(widsT, lens_p, emb, x_out, acc, idx_v, lens_v):
    c = lax.axis_index("c")
    s = lax.axis_index("s")
    base = (s * 2 + c) * PT

    def passes(l, add):
        for cc in range(NCA):
            pltpu.sync_copy(widsT.at[pl.ds(l * NP + base + cc * CA, CA)], idx_v)
            pltpu.sync_copy(emb.at[idx_v], acc.at[pl.ds(cc * CA, CA), :], add=add)

    passes(0, False)

    def lbody(l, carry):
        passes(l, True)
        return carry

    lax.fori_loop(1, L, lbody, 0)

    pltpu.sync_copy(lens_p.at[pl.ds(base, PT)], lens_v)
    ones = jnp.full((16,), 1.0, jnp.float32)

    def nbody(g, carry):
        inv = ones / lens_v[pl.ds(g * 16, 16)]
        for i in range(16):
            sv = inv[i]
            n = g * 16 + i
            for q in range(D // 16):
                acc[n, pl.ds(q * 16, 16)] = acc[n, pl.ds(q * 16, 16)] * sv
        return carry

    lax.fori_loop(0, PT // 16, nbody, 0)

    pltpu.sync_copy(acc.at[:, pl.ds(0, H)], x_out.at[pl.ds(base, PT)])
    pltpu.sync_copy(acc.at[:, pl.ds(H, H)], x_out.at[pl.ds(NP + base, PT)])


_sc_params = pltpu.CompilerParams(use_tc_tiling_on_sc=False,
                                  needs_layout_passes=False)

_pool = pl.kernel(
    _pool_body,
    out_type=jax.ShapeDtypeStruct((2 * NP, H), jnp.float32),
    mesh=plsc.VectorSubcoreMesh(core_axis_name="c", subcore_axis_name="s",
                                num_cores=2, num_subcores=TPS),
    compiler_params=_sc_params,
    scratch_types=[
        pltpu.VMEM((PT, D), jnp.float32),
        pltpu.VMEM((CA,), jnp.int32),
        pltpu.VMEM((PT,), jnp.float32),
    ],
)


def _zero_buf(buf, rows):
    z = jnp.zeros((16,), jnp.float32)

    def zbody(r, carry):
        buf[r, pl.ds(0, 16)] = z
        buf[r, pl.ds(16, 16)] = z
        return carry

    lax.fori_loop(0, rows, zbody, 0)


def _agg_factory(compute_deg):
    def body(*refs):
        if compute_deg:
            (x_cat, edges, ssum_out, invdeg_out, degx, acc_sh,
             src_v, dst_v, src_adj, rows_v, out_buf, deg_part, degbuf,
             degld, sem) = refs
        else:
            (x_cat, edges, invdeg_in, ssum_out, acc_sh,
             src_v, dst_v, src_adj, rows_v, out_buf, degbuf, sem) = refs

        c = lax.axis_index("c")
        s = lax.axis_index("s")
        coff = c * NP
        ones = jnp.full((16,), 1.0, jnp.float32)

        _zero_buf(out_buf, CHN)

        def zc(cc, carry):
            pltpu.sync_copy(out_buf, acc_sh.at[pl.ds(s * PS + cc * CHN, CHN)])
            return carry

        lax.fori_loop(0, PS // CHN, zc, 0)
        plsc.subcore_barrier()

        nmy = jnp.where(s < (NCB % TPS), NCB // TPS + 1, NCB // TPS)

        @pl.loop(0, nmy)
        def _(i):
            off = (s + i * TPS) * CB
            pltpu.sync_copy(edges.at[pl.ds(off, CB)], src_v)
            pltpu.sync_copy(edges.at[pl.ds(E + off, CB)], dst_v)
            for j in range(CB // 16):
                sj = src_v[pl.ds(j * 16, 16)]
                src_adj[pl.ds(j * 16, 16)] = sj + coff
            pltpu.async_copy(x_cat.at[src_adj], rows_v, sem).wait()
            pltpu.async_copy(rows_v, acc_sh.at[dst_v], sem, add=True).wait()

        if compute_deg:
            for p in range(PT // DPR):
                lo = p * (DPR * 32)
                _zero_buf(deg_part, DPR)

                def dbody(i, carry):
                    off = (s + i * TPS) * CB
                    pltpu.sync_copy(edges.at[pl.ds(E + off, CB)], dst_v)
                    for j in range(CB // 16):
                        dj = dst_v[pl.ds(j * 16, 16)] - lo
                        m = (dj >= 0) & (dj < DPR * 32)
                        plsc.addupdate_scatter(
                            deg_part,
                            [lax.shift_right_logical(dj, 5),
                             lax.bitwise_and(dj, 31)],
                            ones, mask=m)
                    return carry

                lax.fori_loop(0, nmy, dbody, 0)

                def pbody(j, carry):
                    for r in range(DGR):
                        for q in range(2):
                            degld[r, pl.ds(q * 16, 16)] = (
                                deg_part[j * DGR + r, pl.ds(q * 16, 16)])
                    pltpu.sync_copy(degld, degx.at[c, s, p * (DPR // DGR) + j])
                    return carry

                lax.fori_loop(0, DPR // DGR, pbody, 0)
        plsc.subcore_barrier()

        if compute_deg:
            pltpu.sync_copy(degx.at[c, 0, s], degbuf)

            def tbody(t, carry):
                pltpu.sync_copy(degx.at[c, t, s], degld)
                for r in range(DGR):
                    for q in range(2):
                        degbuf[r, pl.ds(q * 16, 16)] = (
                            degbuf[r, pl.ds(q * 16, 16)]
                            + degld[r, pl.ds(q * 16, 16)])
                return carry

            lax.fori_loop(1, TPS, tbody, 0)

            def ibody(r, carry):
                for q in range(2):
                    dv = degbuf[r, pl.ds(q * 16, 16)]
                    degbuf[r, pl.ds(q * 16, 16)] = ones / jnp.maximum(dv, ones)
                return carry

            lax.fori_loop(0, DGR, ibody, 0)

            @pl.when(c == 0)
            def _():
                pltpu.sync_copy(degbuf, invdeg_out.at[s])
        else:
            pltpu.sync_copy(invdeg_in.at[s], degbuf)

        def xbody(cc, carry):
            row0 = s * PS + cc * CHN
            pltpu.sync_copy(acc_sh.at[pl.ds(row0, CHN)], out_buf)
            for q in range(CHN // 32):
                for half in range(2):
                    iv = degbuf[cc * (CHN // 32) + q, pl.ds(half * 16, 16)]
                    for i in range(16):
                        sv = iv[i]
                        n = q * 32 + half * 16 + i
                        out_buf[n, pl.ds(0, 16)] = out_buf[n, pl.ds(0, 16)] * sv
                        out_buf[n, pl.ds(16, 16)] = out_buf[n, pl.ds(16, 16)] * sv
            pltpu.sync_copy(out_buf, ssum_out.at[pl.ds(coff + row0, CHN)])
            return carry

        lax.fori_loop(0, PS // CHN, xbody, 0)

    return body


_sc_mesh = plsc.VectorSubcoreMesh(core_axis_name="c", subcore_axis_name="s",
                                  num_cores=2, num_subcores=TPS)

_agg_deg = pl.kernel(
    _agg_factory(True),
    out_type=(jax.ShapeDtypeStruct((2 * NP, H), jnp.float32),
              jax.ShapeDtypeStruct((TPS, DGR, 32), jnp.float32),
              jax.ShapeDtypeStruct((2, TPS, TPS, DGR, 32), jnp.float32)),
    mesh=_sc_mesh,
    compiler_params=_sc_params,
    scratch_types=[
        pltpu.VMEM_SHARED((NP, H), jnp.float32),
        pltpu.VMEM((CB,), jnp.int32),
        pltpu.VMEM((CB,), jnp.int32),
        pltpu.VMEM((CB,), jnp.int32),
        pltpu.VMEM((CB, H), jnp.float32),
        pltpu.VMEM((CHN, 32), jnp.float32),
        pltpu.VMEM((DPR, 32), jnp.float32),
        pltpu.VMEM((DGR, 32), jnp.float32),
        pltpu.VMEM((DGR, 32), jnp.float32),
        pltpu.SemaphoreType.DMA,
    ],
)

_agg = pl.kernel(
    _agg_factory(False),
    out_type=jax.ShapeDtypeStruct((2 * NP, H), jnp.float32),
    mesh=_sc_mesh,
    compiler_params=_sc_params,
    scratch_types=[
        pltpu.VMEM_SHARED((NP, H), jnp.float32),
        pltpu.VMEM((CB,), jnp.int32),
        pltpu.VMEM((CB,), jnp.int32),
        pltpu.VMEM((CB,), jnp.int32),
        pltpu.VMEM((CB, H), jnp.float32),
        pltpu.VMEM((CHN, 32), jnp.float32),
        pltpu.VMEM((DGR, 32), jnp.float32),
        pltpu.SemaphoreType.DMA,
    ],
)

BLK = 512
GR = NP // BLK


def _dense_factory(lrelu, split_out):
    def body(xl, xh, nl, nh, ws, wn, bb, out):
        x = jnp.concatenate([xl[...], xh[...]], axis=1)
        hn = jnp.concatenate([nl[...], nh[...]], axis=1)
        y = (jnp.dot(x, ws[...], preferred_element_type=jnp.float32)
             + jnp.dot(hn, wn[...], preferred_element_type=jnp.float32)
             + bb[...])
        if lrelu:
            y = jnp.where(y > 0, y, y * 0.01)
        if split_out:
            out[0] = y[:, :H]
            out[1] = y[:, H:]
        else:
            out[...] = y

    return body


_dense_in_specs = [
    pl.BlockSpec((BLK, H), lambda i: (i, 0)),
    pl.BlockSpec((BLK, H), lambda i: (i + GR, 0)),
    pl.BlockSpec((BLK, H), lambda i: (i, 0)),
    pl.BlockSpec((BLK, H), lambda i: (i + GR, 0)),
    pl.BlockSpec((D, D), lambda i: (0, 0)),
    pl.BlockSpec((D, D), lambda i: (0, 0)),
    pl.BlockSpec((1, D), lambda i: (0, 0)),
]

_dense1 = pl.pallas_call(
    _dense_factory(True, True),
    grid=(GR,),
    in_specs=_dense_in_specs,
    out_specs=pl.BlockSpec((2, BLK, H), lambda i: (0, i, 0)),
    out_shape=jax.ShapeDtypeStruct((2, NP, H), jnp.float32),
)

_dense2 = pl.pallas_call(
    _dense_factory(False, False),
    grid=(GR,),
    in_specs=_dense_in_specs,
    out_specs=pl.BlockSpec((BLK, D), lambda i: (i, 0)),
    out_shape=jax.ShapeDtypeStruct((NP, D), jnp.float32),
)


def kernel(wids, lens, edge_index, word_emb, W_self1, W_neigh1, b1,
           W_self2, W_neigh2, b2):
    wids32 = wids.astype(jnp.int32)
    widsT = jnp.pad(wids32, ((0, NP - N), (0, 0))).T.reshape(-1)
    lens_p = jnp.pad(lens.astype(jnp.float32), (0, NP - N), constant_values=1.0)
    edges = edge_index.astype(jnp.int32).reshape(-1)

    x = _pool(widsT, lens_p, word_emb.astype(jnp.float32))
    hn1, invdeg, _unused_degx = _agg_deg(x, edges)
    h1 = _dense1(x, x, hn1, hn1, W_self1, W_neigh1,
                 b1.reshape(1, D)).reshape(2 * NP, H)
    hn2 = _agg(h1, edges, invdeg)
    out = _dense2(h1, h1, hn2, hn2, W_self2, W_neigh2, b2.reshape(1, D))
    return out[:N]


# speedup vs baseline: 3.7566x; 1.0959x over previous
import jax
import jax.numpy as jnp
from jax import lax
from jax.experimental import pallas as pl
from jax.experimental.pallas import tpu as pltpu
from jax.experimental.pallas import tpu_sc as plsc

N = 50000
E = 800000
D = 64
L = 20
H = 32
NW = 32
PT = 1568
NP = NW * PT
CA = 112
NCA = PT // CA
CB = 128
NCB = E // CB
TPS = 16
PS = NP // TPS
DGR = 98
CHN = 32
DPR = 196


def _pool_body(widsT, lens_p, emb, x_out, acc, idx_row, lens_v, sem):
    c = lax.axis_index("c")
    s = lax.axis_index("s")
    base = (s * 2 + c) * PT

    def passes(l, add):
        pltpu.sync_copy(widsT.at[pl.ds(l * NP + base, PT)], idx_row)
        for cc in range(NCA):
            pltpu.async_copy(emb.at[idx_row.at[pl.ds(cc * CA, CA)]],
                             acc.at[pl.ds(cc * CA, CA), :], sem, add=add)
        for cc in range(NCA):
            pltpu.make_async_copy(emb.at[pl.ds(0, CA)],
                                  acc.at[pl.ds(0, CA), :], sem).wait()

    passes(0, False)

    @pl.loop(1, L)
    def _(l):
        passes(l, True)

    pltpu.sync_copy(lens_p.at[pl.ds(base, PT)], lens_v)
    ones = jnp.full((16,), 1.0, jnp.float32)

    def nbody(g, carry):
        inv = ones / lens_v[pl.ds(g * 16, 16)]
        for i in range(16):
            sv = inv[i]
            n = g * 16 + i
            for q in range(D // 16):
                acc[n, pl.ds(q * 16, 16)] = acc[n, pl.ds(q * 16, 16)] * sv
        return carry

    lax.fori_loop(0, PT // 16, nbody, 0)

    pltpu.sync_copy(acc.at[:, pl.ds(0, H)], x_out.at[pl.ds(base, PT)])
    pltpu.sync_copy(acc.at[:, pl.ds(H, H)], x_out.at[pl.ds(NP + base, PT)])


_sc_params = pltpu.CompilerParams(use_tc_tiling_on_sc=False,
                                  needs_layout_passes=False)

_pool = pl.kernel(
    _pool_body,
    out_type=jax.ShapeDtypeStruct((2 * NP, H), jnp.float32),
    mesh=plsc.VectorSubcoreMesh(core_axis_name="c", subcore_axis_name="s",
                                num_cores=2, num_subcores=TPS),
    compiler_params=_sc_params,
    scratch_types=[
        pltpu.VMEM((PT, D), jnp.float32),
        pltpu.VMEM((PT,), jnp.int32),
        pltpu.VMEM((PT,), jnp.float32),
        pltpu.SemaphoreType.DMA,
    ],
)


def _zero_buf(buf, rows):
    z = jnp.zeros((16,), jnp.float32)

    def zbody(r, carry):
        buf[r, pl.ds(0, 16)] = z
        buf[r, pl.ds(16, 16)] = z
        return carry

    lax.fori_loop(0, rows, zbody, 0)


def _agg_factory(compute_deg):
    def body(*refs):
        if compute_deg:
            (x_cat, edges, ssum_out, invdeg_out, degx, acc_sh,
             src_v, dst_v, src_adj, rows_v, out_buf, deg_part, degbuf,
             degld, sem_g, sem_s) = refs
        else:
            (x_cat, edges, invdeg_in, ssum_out, acc_sh,
             src_v, dst_v, src_adj, rows_v, out_buf, degbuf,
             sem_g, sem_s) = refs

        c = lax.axis_index("c")
        s = lax.axis_index("s")
        coff = c * NP
        ones = jnp.full((16,), 1.0, jnp.float32)

        _zero_buf(out_buf, CHN)

        def zc(cc, carry):
            pltpu.sync_copy(out_buf, acc_sh.at[pl.ds(s * PS + cc * CHN, CHN)])
            return carry

        lax.fori_loop(0, PS // CHN, zc, 0)
        plsc.subcore_barrier()

        nmy = jnp.where(s < (NCB % TPS), NCB // TPS + 1, NCB // TPS)

        def load_chunk(ci, slot):
            off = (s + ci * TPS) * CB
            pltpu.sync_copy(edges.at[pl.ds(off, CB)], src_v.at[slot])
            pltpu.sync_copy(edges.at[pl.ds(E + off, CB)], dst_v.at[slot])
            for j in range(CB // 16):
                sj = src_v[slot, pl.ds(j * 16, 16)]
                src_adj[slot, pl.ds(j * 16, 16)] = sj + coff

        load_chunk(0, 0)
        pltpu.async_copy(x_cat.at[src_adj.at[0]], rows_v.at[0], sem_g)

        @pl.loop(0, nmy)
        def _(i):
            b = lax.rem(i, 2)
            nb = 1 - b

            @pl.when(i + 1 < nmy)
            def _():
                load_chunk(i + 1, nb)

            pltpu.make_async_copy(x_cat.at[pl.ds(0, CB)],
                                  rows_v.at[b], sem_g).wait()

            @pl.when(i >= 1)
            def _():
                pltpu.make_async_copy(x_cat.at[pl.ds(0, CB)],
                                      rows_v.at[nb], sem_s).wait()

            @pl.when(i + 1 < nmy)
            def _():
                pltpu.async_copy(x_cat.at[src_adj.at[nb]], rows_v.at[nb],
                                 sem_g)

            pltpu.async_copy(rows_v.at[b], acc_sh.at[dst_v.at[b]], sem_s,
                             add=True)

        pltpu.make_async_copy(x_cat.at[pl.ds(0, CB)], rows_v.at[0],
                              sem_s).wait()

        if compute_deg:
            for p in range(PT // DPR):
                lo = p * (DPR * 32)
                _zero_buf(deg_part, DPR)
                pltpu.sync_copy(edges.at[pl.ds(E + s * CB, CB)], dst_v.at[0])

                @pl.loop(0, nmy)
                def _(i):
                    b = lax.rem(i, 2)
                    nb = 1 - b

                    @pl.when(i + 1 < nmy)
                    def _():
                        off = (s + (i + 1) * TPS) * CB
                        pltpu.async_copy(edges.at[pl.ds(E + off, CB)],
                                         dst_v.at[nb], sem_g)

                    for j in range(CB // 16):
                        dj = dst_v[b, pl.ds(j * 16, 16)] - lo
                        m = (dj >= 0) & (dj < DPR * 32)
                        plsc.addupdate_scatter(
                            deg_part,
                            [lax.shift_right_logical(dj, 5),
                             lax.bitwise_and(dj, 31)],
                            ones, mask=m)

                    @pl.when(i + 1 < nmy)
                    def _():
                        pltpu.make_async_copy(edges.at[pl.ds(0, CB)],
                                              dst_v.at[nb], sem_g).wait()

                def pbody(j, carry):
                    for r in range(DGR):
                        for q in range(2):
                            degld[r, pl.ds(q * 16, 16)] = (
                                deg_part[j * DGR + r, pl.ds(q * 16, 16)])
                    pltpu.sync_copy(degld, degx.at[c, s, p * (DPR // DGR) + j])
                    return carry

                lax.fori_loop(0, DPR // DGR, pbody, 0)
        plsc.subcore_barrier()

        if compute_deg:
            pltpu.sync_copy(degx.at[c, 0, s], degbuf)

            def tbody(t, carry):
                pltpu.sync_copy(degx.at[c, t, s], degld)
                for r in range(DGR):
                    for q in range(2):
                        degbuf[r, pl.ds(q * 16, 16)] = (
                            degbuf[r, pl.ds(q * 16, 16)]
                            + degld[r, pl.ds(q * 16, 16)])
                return carry

            lax.fori_loop(1, TPS, tbody, 0)

            def ibody(r, carry):
                for q in range(2):
                    dv = degbuf[r, pl.ds(q * 16, 16)]
                    degbuf[r, pl.ds(q * 16, 16)] = ones / jnp.maximum(dv, ones)
                return carry

            lax.fori_loop(0, DGR, ibody, 0)

            @pl.when(c == 0)
            def _():
                pltpu.sync_copy(degbuf, invdeg_out.at[s])
        else:
            pltpu.sync_copy(invdeg_in.at[s], degbuf)

        def xbody(cc, carry):
            row0 = s * PS + cc * CHN
            pltpu.sync_copy(acc_sh.at[pl.ds(row0, CHN)], out_buf)
            for q in range(CHN // 32):
                for half in range(2):
                    iv = degbuf[cc * (CHN // 32) + q, pl.ds(half * 16, 16)]
                    for i in range(16):
                        sv = iv[i]
                        n = q * 32 + half * 16 + i
                        out_buf[n, pl.ds(0, 16)] = out_buf[n, pl.ds(0, 16)] * sv
                        out_buf[n, pl.ds(16, 16)] = out_buf[n, pl.ds(16, 16)] * sv
            pltpu.sync_copy(out_buf, ssum_out.at[pl.ds(coff + row0, CHN)])
            return carry

        lax.fori_loop(0, PS // CHN, xbody, 0)

    return body


_sc_mesh = plsc.VectorSubcoreMesh(core_axis_name="c", subcore_axis_name="s",
                                  num_cores=2, num_subcores=TPS)

_agg_deg = pl.kernel(
    _agg_factory(True),
    out_type=(jax.ShapeDtypeStruct((2 * NP, H), jnp.float32),
              jax.ShapeDtypeStruct((TPS, DGR, 32), jnp.float32),
              jax.ShapeDtypeStruct((2, TPS, TPS, DGR, 32), jnp.float32)),
    mesh=_sc_mesh,
    compiler_params=_sc_params,
    scratch_types=[
        pltpu.VMEM_SHARED((NP, H), jnp.float32),
        pltpu.VMEM((2, CB), jnp.int32),
        pltpu.VMEM((2, CB), jnp.int32),
        pltpu.VMEM((2, CB), jnp.int32),
        pltpu.VMEM((2, CB, H), jnp.float32),
        pltpu.VMEM((CHN, 32), jnp.float32),
        pltpu.VMEM((DPR, 32), jnp.float32),
        pltpu.VMEM((DGR, 32), jnp.float32),
        pltpu.VMEM((DGR, 32), jnp.float32),
        pltpu.SemaphoreType.DMA,
        pltpu.SemaphoreType.DMA,
    ],
)

_agg = pl.kernel(
    _agg_factory(False),
    out_type=jax.ShapeDtypeStruct((2 * NP, H), jnp.float32),
    mesh=_sc_mesh,
    compiler_params=_sc_params,
    scratch_types=[
        pltpu.VMEM_SHARED((NP, H), jnp.float32),
        pltpu.VMEM((2, CB), jnp.int32),
        pltpu.VMEM((2, CB), jnp.int32),
        pltpu.VMEM((2, CB), jnp.int32),
        pltpu.VMEM((2, CB, H), jnp.float32),
        pltpu.VMEM((CHN, 32), jnp.float32),
        pltpu.VMEM((DGR, 32), jnp.float32),
        pltpu.SemaphoreType.DMA,
        pltpu.SemaphoreType.DMA,
    ],
)

BLK = 512
GR = NP // BLK


def _dense_factory(lrelu, split_out):
    def body(xl, xh, nl, nh, ws, wn, bb, out):
        x = jnp.concatenate([xl[...], xh[...]], axis=1)
        hn = jnp.concatenate([nl[...], nh[...]], axis=1)
        y = (jnp.dot(x, ws[...], preferred_element_type=jnp.float32)
             + jnp.dot(hn, wn[...], preferred_element_type=jnp.float32)
             + bb[...])
        if lrelu:
            y = jnp.where(y > 0, y, y * 0.01)
        if split_out:
            out[0] = y[:, :H]
            out[1] = y[:, H:]
        else:
            out[...] = y

    return body


_dense_in_specs = [
    pl.BlockSpec((BLK, H), lambda i: (i, 0)),
    pl.BlockSpec((BLK, H), lambda i: (i + GR, 0)),
    pl.BlockSpec((BLK, H), lambda i: (i, 0)),
    pl.BlockSpec((BLK, H), lambda i: (i + GR, 0)),
    pl.BlockSpec((D, D), lambda i: (0, 0)),
    pl.BlockSpec((D, D), lambda i: (0, 0)),
    pl.BlockSpec((1, D), lambda i: (0, 0)),
]

_dense1 = pl.pallas_call(
    _dense_factory(True, True),
    grid=(GR,),
    in_specs=_dense_in_specs,
    out_specs=pl.BlockSpec((2, BLK, H), lambda i: (0, i, 0)),
    out_shape=jax.ShapeDtypeStruct((2, NP, H), jnp.float32),
)

_dense2 = pl.pallas_call(
    _dense_factory(False, False),
    grid=(GR,),
    in_specs=_dense_in_specs,
    out_specs=pl.BlockSpec((BLK, D), lambda i: (i, 0)),
    out_shape=jax.ShapeDtypeStruct((NP, D), jnp.float32),
)


def kernel(wids, lens, edge_index, word_emb, W_self1, W_neigh1, b1,
           W_self2, W_neigh2, b2):
    wids32 = wids.astype(jnp.int32)
    widsT = jnp.pad(wids32, ((0, NP - N), (0, 0))).T.reshape(-1)
    lens_p = jnp.pad(lens.astype(jnp.float32), (0, NP - N), constant_values=1.0)
    edges = edge_index.astype(jnp.int32).reshape(-1)

    x = _pool(widsT, lens_p, word_emb.astype(jnp.float32))
    hn1, invdeg, _unused_degx = _agg_deg(x, edges)
    h1 = _dense1(x, x, hn1, hn1, W_self1, W_neigh1,
                 b1.reshape(1, D)).reshape(2 * NP, H)
    hn2 = _agg(h1, edges, invdeg)
    out = _dense2(h1, h1, hn2, hn2, W_self2, W_neigh2, b2.reshape(1, D))
    return out[:N]


# speedup vs baseline: 6.4902x; 1.7277x over previous
import jax
import jax.numpy as jnp
from jax import lax
from jax.experimental import pallas as pl
from jax.experimental.pallas import tpu as pltpu
from jax.experimental.pallas import tpu_sc as plsc

N = 50000
E = 800000
D = 64
L = 20
H = 32
NW = 32
PT = 1568
NP = NW * PT
CA = 112
NCA = PT // CA
CB = 128
NCB = E // CB
TPS = 16
PS = NP // TPS
DGR = 98
CHN = 112


def _pool_body(widsT, lens_p, emb, x_out, acc, idx_row, lens_v, sem):
    c = lax.axis_index("c")
    s = lax.axis_index("s")
    base = (s * 2 + c) * PT

    def passes(l, add):
        pltpu.sync_copy(widsT.at[pl.ds(l * NP + base, PT)], idx_row)
        for cc in range(NCA):
            pltpu.async_copy(emb.at[idx_row.at[pl.ds(cc * CA, CA)]],
                             acc.at[pl.ds(cc * CA, CA), :], sem, add=add)
        for cc in range(NCA):
            pltpu.make_async_copy(emb.at[pl.ds(0, CA)],
                                  acc.at[pl.ds(0, CA), :], sem).wait()

    passes(0, False)

    @pl.loop(1, L)
    def _(l):
        passes(l, True)

    pltpu.sync_copy(lens_p.at[pl.ds(base, PT)], lens_v)
    ones = jnp.full((16,), 1.0, jnp.float32)

    def nbody(g, carry):
        inv = ones / lens_v[pl.ds(g * 16, 16)]
        for i in range(16):
            sv = inv[i]
            n = g * 16 + i
            for q in range(D // 16):
                acc[n, pl.ds(q * 16, 16)] = acc[n, pl.ds(q * 16, 16)] * sv
        return carry

    lax.fori_loop(0, PT // 16, nbody, 0)

    pltpu.sync_copy(acc.at[:, pl.ds(0, H)], x_out.at[pl.ds(base, PT)])
    pltpu.sync_copy(acc.at[:, pl.ds(H, H)], x_out.at[pl.ds(NP + base, PT)])


_sc_params = pltpu.CompilerParams(use_tc_tiling_on_sc=False,
                                  needs_layout_passes=False)

_pool = pl.kernel(
    _pool_body,
    out_type=jax.ShapeDtypeStruct((2 * NP, H), jnp.float32),
    mesh=plsc.VectorSubcoreMesh(core_axis_name="c", subcore_axis_name="s",
                                num_cores=2, num_subcores=TPS),
    compiler_params=_sc_params,
    scratch_types=[
        pltpu.VMEM((PT, D), jnp.float32),
        pltpu.VMEM((PT,), jnp.int32),
        pltpu.VMEM((PT,), jnp.float32),
        pltpu.SemaphoreType.DMA,
    ],
)


def _zero_buf(buf, rows):
    z = jnp.zeros((16,), jnp.float32)

    def zbody(r, carry):
        buf[r, pl.ds(0, 16)] = z
        buf[r, pl.ds(16, 16)] = z
        return carry

    lax.fori_loop(0, rows, zbody, 0)


def _agg_factory(compute_deg):
    def body(*refs):
        if compute_deg:
            (x_cat, edges, ssum_out, invdeg_out, acc_sh,
             src_v, dst_v, src_adj, rows_v, out_buf, ivbuf, zbuf,
             sem_g, sem_s) = refs
            invdeg = invdeg_out
        else:
            (x_cat, edges, invdeg_in, ssum_out, acc_sh,
             src_v, dst_v, src_adj, rows_v, out_buf, ivbuf, zbuf,
             sem_g, sem_s) = refs
            invdeg = invdeg_in

        c = lax.axis_index("c")
        s = lax.axis_index("s")
        coff = c * NP
        one = jnp.full((16,), 1.0, jnp.float32)

        _zero_buf(zbuf, CHN)

        @pl.loop(0, PS // CHN)
        def _(cc):
            pltpu.sync_copy(zbuf, acc_sh.at[pl.ds(s * PS + cc * CHN, CHN)])

        plsc.subcore_barrier()

        nmy = jnp.where(s < (NCB % TPS), NCB // TPS + 1, NCB // TPS)

        if compute_deg:
            for r in range(CB):
                rows_v[0, r, pl.ds(0, 16)] = one
                rows_v[0, r, pl.ds(16, 16)] = one
            pltpu.sync_copy(edges.at[pl.ds(E + s * CB, CB)], dst_v.at[0])

            @pl.loop(0, nmy)
            def _(i):
                b = lax.rem(i, 2)
                nb = 1 - b

                @pl.when(i + 1 < nmy)
                def _():
                    off = (s + (i + 1) * TPS) * CB
                    pltpu.async_copy(edges.at[pl.ds(E + off, CB)],
                                     dst_v.at[nb], sem_g)

                @pl.when(i >= 1)
                def _():
                    pltpu.make_async_copy(x_cat.at[pl.ds(0, CB)],
                                          rows_v.at[0], sem_s).wait()

                pltpu.async_copy(rows_v.at[0], acc_sh.at[dst_v.at[b]], sem_s,
                                 add=True)

                @pl.when(i + 1 < nmy)
                def _():
                    pltpu.make_async_copy(edges.at[pl.ds(0, CB)],
                                          dst_v.at[nb], sem_g).wait()

            pltpu.make_async_copy(x_cat.at[pl.ds(0, CB)], rows_v.at[0],
                                  sem_s).wait()
            plsc.subcore_barrier()

            @pl.loop(0, PS // CHN)
            def _(cc):
                row0 = s * PS + cc * CHN
                pltpu.sync_copy(acc_sh.at[pl.ds(row0, CHN)], ivbuf)
                for r in range(CHN):
                    for q in range(2):
                        dv = ivbuf[r, pl.ds(q * 16, 16)]
                        ivbuf[r, pl.ds(q * 16, 16)] = one / jnp.maximum(dv, one)
                pltpu.sync_copy(ivbuf, invdeg.at[pl.ds(row0, CHN)])
                pltpu.sync_copy(zbuf, acc_sh.at[pl.ds(row0, CHN)])

            plsc.subcore_barrier()

        def load_chunk(ci, slot):
            off = (s + ci * TPS) * CB
            pltpu.sync_copy(edges.at[pl.ds(off, CB)], src_v.at[slot])
            pltpu.sync_copy(edges.at[pl.ds(E + off, CB)], dst_v.at[slot])
            for j in range(CB // 16):
                sj = src_v[slot, pl.ds(j * 16, 16)]
                src_adj[slot, pl.ds(j * 16, 16)] = sj + coff

        load_chunk(0, 0)
        pltpu.async_copy(x_cat.at[src_adj.at[0]], rows_v.at[0], sem_g)

        @pl.loop(0, nmy)
        def _(i):
            b = lax.rem(i, 2)
            nb = 1 - b

            @pl.when(i + 1 < nmy)
            def _():
                load_chunk(i + 1, nb)

            pltpu.make_async_copy(x_cat.at[pl.ds(0, CB)],
                                  rows_v.at[b], sem_g).wait()

            @pl.when(i >= 1)
            def _():
                pltpu.make_async_copy(x_cat.at[pl.ds(0, CB)],
                                      rows_v.at[nb], sem_s).wait()

            @pl.when(i + 1 < nmy)
            def _():
                pltpu.async_copy(x_cat.at[src_adj.at[nb]], rows_v.at[nb],
                                 sem_g)

            pltpu.async_copy(rows_v.at[b], acc_sh.at[dst_v.at[b]], sem_s,
                             add=True)

        pltpu.make_async_copy(x_cat.at[pl.ds(0, CB)], rows_v.at[0],
                              sem_s).wait()
        plsc.subcore_barrier()

        @pl.loop(0, PS // CHN)
        def _(cc):
            row0 = s * PS + cc * CHN
            pltpu.sync_copy(acc_sh.at[pl.ds(row0, CHN)], out_buf)
            pltpu.sync_copy(invdeg.at[pl.ds(row0, CHN)], ivbuf)
            for r in range(CHN):
                for q in range(2):
                    out_buf[r, pl.ds(q * 16, 16)] = (
                        out_buf[r, pl.ds(q * 16, 16)]
                        * ivbuf[r, pl.ds(q * 16, 16)])
            pltpu.sync_copy(out_buf, ssum_out.at[pl.ds(coff + row0, CHN)])

    return body


_sc_mesh = plsc.VectorSubcoreMesh(core_axis_name="c", subcore_axis_name="s",
                                  num_cores=2, num_subcores=TPS)

_agg_scratch = [
    pltpu.VMEM_SHARED((NP, H), jnp.float32),
    pltpu.VMEM((2, CB), jnp.int32),
    pltpu.VMEM((2, CB), jnp.int32),
    pltpu.VMEM((2, CB), jnp.int32),
    pltpu.VMEM((2, CB, H), jnp.float32),
    pltpu.VMEM((CHN, 32), jnp.float32),
    pltpu.VMEM((CHN, 32), jnp.float32),
    pltpu.VMEM((CHN, 32), jnp.float32),
    pltpu.SemaphoreType.DMA,
    pltpu.SemaphoreType.DMA,
]

_agg_deg = pl.kernel(
    _agg_factory(True),
    out_type=(jax.ShapeDtypeStruct((2 * NP, H), jnp.float32),
              jax.ShapeDtypeStruct((NP, 32), jnp.float32)),
    mesh=_sc_mesh,
    compiler_params=_sc_params,
    scratch_types=_agg_scratch,
)

_agg = pl.kernel(
    _agg_factory(False),
    out_type=jax.ShapeDtypeStruct((2 * NP, H), jnp.float32),
    mesh=_sc_mesh,
    compiler_params=_sc_params,
    scratch_types=_agg_scratch,
)

BLK = 512
GR = NP // BLK


def _dense_factory(lrelu, split_out):
    def body(xl, xh, nl, nh, ws, wn, bb, out):
        x = jnp.concatenate([xl[...], xh[...]], axis=1)
        hn = jnp.concatenate([nl[...], nh[...]], axis=1)
        y = (jnp.dot(x, ws[...], preferred_element_type=jnp.float32)
             + jnp.dot(hn, wn[...], preferred_element_type=jnp.float32)
             + bb[...])
        if lrelu:
            y = jnp.where(y > 0, y, y * 0.01)
        if split_out:
            out[0] = y[:, :H]
            out[1] = y[:, H:]
        else:
            out[...] = y

    return body


_dense_in_specs = [
    pl.BlockSpec((BLK, H), lambda i: (i, 0)),
    pl.BlockSpec((BLK, H), lambda i: (i + GR, 0)),
    pl.BlockSpec((BLK, H), lambda i: (i, 0)),
    pl.BlockSpec((BLK, H), lambda i: (i + GR, 0)),
    pl.BlockSpec((D, D), lambda i: (0, 0)),
    pl.BlockSpec((D, D), lambda i: (0, 0)),
    pl.BlockSpec((1, D), lambda i: (0, 0)),
]

_dense1 = pl.pallas_call(
    _dense_factory(True, True),
    grid=(GR,),
    in_specs=_dense_in_specs,
    out_specs=pl.BlockSpec((2, BLK, H), lambda i: (0, i, 0)),
    out_shape=jax.ShapeDtypeStruct((2, NP, H), jnp.float32),
)

_dense2 = pl.pallas_call(
    _dense_factory(False, False),
    grid=(GR,),
    in_specs=_dense_in_specs,
    out_specs=pl.BlockSpec((BLK, D), lambda i: (i, 0)),
    out_shape=jax.ShapeDtypeStruct((NP, D), jnp.float32),
)


def kernel(wids, lens, edge_index, word_emb, W_self1, W_neigh1, b1,
           W_self2, W_neigh2, b2):
    wids32 = wids.astype(jnp.int32)
    widsT = jnp.pad(wids32, ((0, NP - N), (0, 0))).T.reshape(-1)
    lens_p = jnp.pad(lens.astype(jnp.float32), (0, NP - N), constant_values=1.0)
    edges = edge_index.astype(jnp.int32).reshape(-1)

    x = _pool(widsT, lens_p, word_emb.astype(jnp.float32))
    hn1, invdeg = _agg_deg(x, edges)
    h1 = _dense1(x, x, hn1, hn1, W_self1, W_neigh1,
                 b1.reshape(1, D)).reshape(2 * NP, H)
    hn2 = _agg(h1, edges, invdeg)
    out = _dense2(h1, h1, hn2, hn2, W_self2, W_neigh2, b2.reshape(1, D))
    return out[:N]
